# Initial kernel scaffold; baseline (speedup 1.0000x reference)
import jax
import jax.numpy as jnp
from jax.experimental import pallas as pl


def kernel(w_sigma, w_rgb, rays):
    raise NotImplementedError("write your pallas kernel here")


# speedup vs baseline: 263.3418x; 263.3418x over previous
import functools

import jax
import jax.numpy as jnp
from jax import lax
from jax.experimental import pallas as pl
from jax.experimental.pallas import tpu as pltpu
from jax.experimental.pallas import tpu_sc as plsc

GRID = 128
MAX_S = 256
N_RAYS = 8192
RPB = 128
NBLK = N_RAYS // RPB
SEG = 16
NSEG = MAX_S // SEG
NGRP = RPB // 16


def _tc_prepass(rays_t_ref, idx_ref, ns_ref, dist_ref):
    ox = rays_t_ref[0:1, :]
    oy = rays_t_ref[1:2, :]
    oz = rays_t_ref[2:3, :]
    dxr = rays_t_ref[3:4, :]
    dyr = rays_t_ref[4:5, :]
    dzr = rays_t_ref[5:6, :]
    nrm = jnp.sqrt(dxr * dxr + dyr * dyr + dzr * dzr)
    dx = dxr / nrm
    dy = dyr / nrm
    dz = dzr / nrm

    big = jnp.float32(1e30)

    def slab(o, d):
        zero = d == 0.0
        safe = jnp.where(zero, 1.0, d)
        i1 = jnp.where(zero, -big, (-1.5 - o) / safe)
        i2 = jnp.where(zero, big, (1.5 - o) / safe)
        nn = jnp.minimum(i1, i2)
        ff = jnp.maximum(i1, i2)
        okax = jnp.logical_or(~zero, (o >= -1.5) & (o <= 1.5))
        return nn, ff, okax

    nnx, ffx, okx = slab(ox, dx)
    nny, ffy, oky = slab(oy, dy)
    nnz, ffz, okz = slab(oz, dz)
    near = jnp.maximum(jnp.maximum(nnx, nny), nnz)
    far = jnp.minimum(jnp.minimum(ffx, ffy), ffz)
    ok = okx & oky & okz
    isect = (near <= far) & ok
    span = far - near
    ns = jnp.where(isect,
                   jnp.minimum(span * 32.0, 256.0).astype(jnp.int32),
                   jnp.int32(0))
    ns_f = jnp.maximum(ns, 1).astype(jnp.float32)
    dist = span / ns_f

    j = lax.broadcasted_iota(jnp.int32, (MAX_S, RPB), 0).astype(jnp.float32)
    t = near + span * (j + 0.5) / ns_f

    def axis_idx(o, d):
        pos = (o + d * t) / 1.5 * 0.5 + 0.5
        return jnp.clip(jnp.floor(pos * GRID).astype(jnp.int32), 0, GRID - 1)

    ix = axis_idx(ox, dx)
    iy = axis_idx(oy, dy)
    iz = axis_idx(oz, dz)
    idx = (ix * GRID + iy) * GRID + iz
    idx_ref[...] = idx.reshape(1, MAX_S, RPB)
    ns_ref[...] = ns.reshape(1, 1, RPB)
    dist_ref[...] = dist.reshape(1, 1, RPB)


def _sc_render(table_ref, idx_hbm, ns_hbm, dist_hbm, c_hbm,
               idx_v, sig_v, ns_v, dist_v, c_v, mv_v, sem):
    cid = lax.axis_index("c")
    sid = lax.axis_index("s")
    wid = sid * 2 + cid

    for i in range(2):
        p = wid * 2 + i
        pltpu.sync_copy(idx_hbm.at[p], idx_v)
        pltpu.sync_copy(ns_hbm.at[p], ns_v)
        pltpu.sync_copy(dist_hbm.at[p], dist_v)

        mv = ns_v[pl.ds(0, 16)]
        for g in range(1, NGRP):
            mv = jnp.maximum(mv, ns_v[pl.ds(g * 16, 16)])
        m = mv[0]
        for l in range(1, 16):
            m = jnp.maximum(m, mv[l])
        nseg = (m + (SEG - 1)) >> 4

        seg_elems = SEG * RPB

        def fire(b, _):
            pltpu.make_async_copy(
                table_ref.at[idx_v.at[pl.ds(b * seg_elems, seg_elems)]],
                sig_v.at[pl.ds(b * seg_elems, seg_elems)],
                sem,
            ).start()
            return 0

        lax.fori_loop(0, nseg, fire, 0)

        def drain(b, _):
            pltpu.make_async_copy(
                table_ref.at[idx_v.at[pl.ds(b * seg_elems, seg_elems)]],
                sig_v.at[pl.ds(b * seg_elems, seg_elems)],
                sem,
            ).wait()
            return 0

        lax.fori_loop(0, nseg, drain, 0)

        nsg = [ns_v[pl.ds(g * 16, 16)] for g in range(NGRP)]
        dsg = [dist_v[pl.ds(g * 16, 16)] for g in range(NGRP)]
        ones = jnp.ones((16,), jnp.float32)

        def body(jj, carry):
            newc = []
            for g in range(NGRP):
                P, C = carry[2 * g], carry[2 * g + 1]
                sig = sig_v[pl.ds(jj * RPB + g * 16, 16)]
                valid = nsg[g] > jj
                s = jnp.maximum(sig, 0.0)
                e = jnp.exp(s * dsg[g])
                a = 1.0 - e
                om = jnp.where(valid, 1.0 - a, 1.0)
                P = P * om
                w = jnp.where(valid, a * P, 0.0)
                C = C * (1.0 + w)
                newc.append(P)
                newc.append(C)
            return tuple(newc)

        carry = lax.fori_loop(0, m, body, tuple([ones] * (2 * NGRP)))
        for g in range(NGRP):
            c_v[pl.ds(g * 16, 16)] = carry[2 * g + 1]
        pltpu.sync_copy(c_v, c_hbm.at[p])


@jax.jit
def kernel(w_sigma, w_rgb, rays):
    del w_rgb
    rays_t = rays.T

    idx, ns3, dist3 = pl.pallas_call(
        _tc_prepass,
        grid=(NBLK,),
        in_specs=[pl.BlockSpec((6, RPB), lambda p: (0, p))],
        out_specs=[
            pl.BlockSpec((1, MAX_S, RPB), lambda p: (p, 0, 0)),
            pl.BlockSpec((1, 1, RPB), lambda p: (p, 0, 0)),
            pl.BlockSpec((1, 1, RPB), lambda p: (p, 0, 0)),
        ],
        out_shape=[
            jax.ShapeDtypeStruct((NBLK, MAX_S, RPB), jnp.int32),
            jax.ShapeDtypeStruct((NBLK, 1, RPB), jnp.int32),
            jax.ShapeDtypeStruct((NBLK, 1, RPB), jnp.float32),
        ],
    )(rays_t)

    table = w_sigma.reshape(GRID * GRID * GRID)
    idx = idx.reshape(NBLK, MAX_S * RPB)
    ns2 = ns3.reshape(NBLK, RPB)
    dist2 = dist3.reshape(NBLK, RPB)

    c2 = pl.kernel(
        _sc_render,
        out_type=jax.ShapeDtypeStruct((NBLK, RPB), jnp.float32),
        mesh=plsc.VectorSubcoreMesh(core_axis_name="c", subcore_axis_name="s"),
        scratch_types=[
            pltpu.VMEM((MAX_S * RPB,), jnp.int32),
            pltpu.VMEM((MAX_S * RPB,), jnp.float32),
            pltpu.VMEM((RPB,), jnp.int32),
            pltpu.VMEM((RPB,), jnp.float32),
            pltpu.VMEM((RPB,), jnp.float32),
            pltpu.VMEM((16,), jnp.int32),
            pltpu.SemaphoreType.DMA,
        ],
    )(table, idx, ns2, dist2)

    c = c2.reshape(N_RAYS)
    return jnp.stack([c, c, c, 1.0 - c], axis=1)


# speedup vs baseline: 284.0999x; 1.0788x over previous
import functools

import jax
import jax.numpy as jnp
from jax import lax
from jax.experimental import pallas as pl
from jax.experimental.pallas import tpu as pltpu
from jax.experimental.pallas import tpu_sc as plsc

GRID = 128
MAX_S = 256
N_RAYS = 8192
RPB = 128
NBLK = N_RAYS // RPB
SEG = 8
NSEG = MAX_S // SEG
NGRP = RPB // 16
CAP_S = 168


def _tc_prepass(rays_t_ref, idx_ref, ns_ref, dist_ref):
    ox = rays_t_ref[0:1, :]
    oy = rays_t_ref[1:2, :]
    oz = rays_t_ref[2:3, :]
    dxr = rays_t_ref[3:4, :]
    dyr = rays_t_ref[4:5, :]
    dzr = rays_t_ref[5:6, :]
    nrm = jnp.sqrt(dxr * dxr + dyr * dyr + dzr * dzr)
    dx = dxr / nrm
    dy = dyr / nrm
    dz = dzr / nrm

    big = jnp.float32(1e30)

    def slab(o, d):
        zero = d == 0.0
        safe = jnp.where(zero, 1.0, d)
        i1 = jnp.where(zero, -big, (-1.5 - o) / safe)
        i2 = jnp.where(zero, big, (1.5 - o) / safe)
        nn = jnp.minimum(i1, i2)
        ff = jnp.maximum(i1, i2)
        okax = jnp.logical_or(~zero, (o >= -1.5) & (o <= 1.5))
        return nn, ff, okax

    nnx, ffx, okx = slab(ox, dx)
    nny, ffy, oky = slab(oy, dy)
    nnz, ffz, okz = slab(oz, dz)
    near = jnp.maximum(jnp.maximum(nnx, nny), nnz)
    far = jnp.minimum(jnp.minimum(ffx, ffy), ffz)
    ok = okx & oky & okz
    isect = (near <= far) & ok
    span = far - near
    ns = jnp.where(isect,
                   jnp.minimum(span * 32.0, 256.0).astype(jnp.int32),
                   jnp.int32(0))
    ns_f = jnp.maximum(ns, 1).astype(jnp.float32)
    dist = span / ns_f

    j = lax.broadcasted_iota(jnp.int32, (CAP_S, RPB), 0).astype(jnp.float32)
    t = near + span * (j + 0.5) / ns_f

    def axis_idx(o, d):
        pos = (o + d * t) / 1.5 * 0.5 + 0.5
        return jnp.clip(jnp.floor(pos * GRID).astype(jnp.int32), 0, GRID - 1)

    ix = axis_idx(ox, dx)
    iy = axis_idx(oy, dy)
    iz = axis_idx(oz, dz)
    idx = (ix * GRID + iy) * GRID + iz
    idx_ref[...] = idx.reshape(1, CAP_S, RPB)
    ns_ref[...] = ns.reshape(1, 1, RPB)
    dist_ref[...] = dist.reshape(1, 1, RPB)


def _sc_render(table_ref, idx_hbm, ns_hbm, dist_hbm, c_hbm,
               idx_v, sig_v, ns_v, dist_v, c_v, mv_v, sem):
    cid = lax.axis_index("c")
    sid = lax.axis_index("s")
    wid = sid * 2 + cid

    for i in range(2):
        p = wid * 2 + i
        pltpu.sync_copy(idx_hbm.at[p], idx_v)
        pltpu.sync_copy(ns_hbm.at[p], ns_v)
        pltpu.sync_copy(dist_hbm.at[p], dist_v)

        mv = ns_v[pl.ds(0, 16)]
        for g in range(1, NGRP):
            mv = jnp.maximum(mv, ns_v[pl.ds(g * 16, 16)])
        m = mv[0]
        for l in range(1, 16):
            m = jnp.maximum(m, mv[l])
        nseg = (m + (SEG - 1)) >> 3

        seg_elems = SEG * RPB

        def fire(b, _):
            pltpu.make_async_copy(
                table_ref.at[idx_v.at[pl.ds(b * seg_elems, seg_elems)]],
                sig_v.at[pl.ds(b * seg_elems, seg_elems)],
                sem,
            ).start()
            return 0

        lax.fori_loop(0, nseg, fire, 0)

        def drain(b, _):
            pltpu.make_async_copy(
                table_ref.at[idx_v.at[pl.ds(b * seg_elems, seg_elems)]],
                sig_v.at[pl.ds(b * seg_elems, seg_elems)],
                sem,
            ).wait()
            return 0

        lax.fori_loop(0, nseg, drain, 0)

        nsg = [ns_v[pl.ds(g * 16, 16)] for g in range(NGRP)]
        dsg = [dist_v[pl.ds(g * 16, 16)] for g in range(NGRP)]
        ones = jnp.ones((16,), jnp.float32)

        def body(jj, carry):
            newc = []
            for g in range(NGRP):
                P, C = carry[2 * g], carry[2 * g + 1]
                sig = sig_v[pl.ds(jj * RPB + g * 16, 16)]
                valid = nsg[g] > jj
                s = jnp.maximum(sig, 0.0)
                e = jnp.exp(s * dsg[g])
                a = 1.0 - e
                om = jnp.where(valid, 1.0 - a, 1.0)
                P = P * om
                w = jnp.where(valid, a * P, 0.0)
                C = C * (1.0 + w)
                newc.append(P)
                newc.append(C)
            return tuple(newc)

        carry = lax.fori_loop(0, m, body, tuple([ones] * (2 * NGRP)))
        for g in range(NGRP):
            c_v[pl.ds(g * 16, 16)] = carry[2 * g + 1]
        pltpu.sync_copy(c_v, c_hbm.at[p])


@jax.jit
def kernel(w_sigma, w_rgb, rays):
    del w_rgb
    rays_t = rays.T

    idx, ns3, dist3 = pl.pallas_call(
        _tc_prepass,
        grid=(NBLK,),
        in_specs=[pl.BlockSpec((6, RPB), lambda p: (0, p))],
        out_specs=[
            pl.BlockSpec((1, CAP_S, RPB), lambda p: (p, 0, 0)),
            pl.BlockSpec((1, 1, RPB), lambda p: (p, 0, 0)),
            pl.BlockSpec((1, 1, RPB), lambda p: (p, 0, 0)),
        ],
        out_shape=[
            jax.ShapeDtypeStruct((NBLK, CAP_S, RPB), jnp.int32),
            jax.ShapeDtypeStruct((NBLK, 1, RPB), jnp.int32),
            jax.ShapeDtypeStruct((NBLK, 1, RPB), jnp.float32),
        ],
    )(rays_t)

    table = w_sigma.reshape(GRID * GRID * GRID)
    idx = idx.reshape(NBLK, CAP_S * RPB)
    ns2 = ns3.reshape(NBLK, RPB)
    dist2 = dist3.reshape(NBLK, RPB)

    c2 = pl.kernel(
        _sc_render,
        out_type=jax.ShapeDtypeStruct((NBLK, RPB), jnp.float32),
        mesh=plsc.VectorSubcoreMesh(core_axis_name="c", subcore_axis_name="s"),
        scratch_types=[
            pltpu.VMEM((CAP_S * RPB,), jnp.int32),
            pltpu.VMEM((CAP_S * RPB,), jnp.float32),
            pltpu.VMEM((RPB,), jnp.int32),
            pltpu.VMEM((RPB,), jnp.float32),
            pltpu.VMEM((RPB,), jnp.float32),
            pltpu.VMEM((16,), jnp.int32),
            pltpu.SemaphoreType.DMA,
        ],
    )(table, idx, ns2, dist2)

    c = c2.reshape(N_RAYS)
    return jnp.stack([c, c, c, 1.0 - c], axis=1)


# speedup vs baseline: 285.7186x; 1.0057x over previous
import functools

import jax
import jax.numpy as jnp
from jax import lax
from jax.experimental import pallas as pl
from jax.experimental.pallas import tpu as pltpu
from jax.experimental.pallas import tpu_sc as plsc

GRID = 128
MAX_S = 256
N_RAYS = 8192
RPB = 128
NBLK = N_RAYS // RPB
SEG = 8
NSEG = MAX_S // SEG
NGRP = RPB // 16
CAP_S = 168


def _tc_prepass(rays_t_ref, idx_ref, ns_ref, dist_ref):
    ox = rays_t_ref[0:1, :]
    oy = rays_t_ref[1:2, :]
    oz = rays_t_ref[2:3, :]
    dxr = rays_t_ref[3:4, :]
    dyr = rays_t_ref[4:5, :]
    dzr = rays_t_ref[5:6, :]
    nrm = jnp.sqrt(dxr * dxr + dyr * dyr + dzr * dzr)
    dx = dxr / nrm
    dy = dyr / nrm
    dz = dzr / nrm

    big = jnp.float32(1e30)

    def slab(o, d):
        zero = d == 0.0
        safe = jnp.where(zero, 1.0, d)
        i1 = jnp.where(zero, -big, (-1.5 - o) / safe)
        i2 = jnp.where(zero, big, (1.5 - o) / safe)
        nn = jnp.minimum(i1, i2)
        ff = jnp.maximum(i1, i2)
        okax = jnp.logical_or(~zero, (o >= -1.5) & (o <= 1.5))
        return nn, ff, okax

    nnx, ffx, okx = slab(ox, dx)
    nny, ffy, oky = slab(oy, dy)
    nnz, ffz, okz = slab(oz, dz)
    near = jnp.maximum(jnp.maximum(nnx, nny), nnz)
    far = jnp.minimum(jnp.minimum(ffx, ffy), ffz)
    ok = okx & oky & okz
    isect = (near <= far) & ok
    span = far - near
    ns = jnp.where(isect,
                   jnp.minimum(span * 32.0, 256.0).astype(jnp.int32),
                   jnp.int32(0))
    ns_f = jnp.maximum(ns, 1).astype(jnp.float32)
    dist = span / ns_f

    j = lax.broadcasted_iota(jnp.int32, (CAP_S, RPB), 0).astype(jnp.float32)
    t = near + span * (j + 0.5) / ns_f

    def axis_idx(o, d):
        pos = (o + d * t) / 1.5 * 0.5 + 0.5
        return jnp.clip(jnp.floor(pos * GRID).astype(jnp.int32), 0, GRID - 1)

    ix = axis_idx(ox, dx)
    iy = axis_idx(oy, dy)
    iz = axis_idx(oz, dz)
    idx = (ix * GRID + iy) * GRID + iz
    idx_ref[...] = idx.reshape(1, CAP_S, RPB)
    ns_ref[...] = ns.reshape(1, 1, RPB)
    dist_ref[...] = dist.reshape(1, 1, RPB)


def _sc_render(table_ref, idx_hbm, ns_hbm, dist_hbm, c_hbm,
               idx_v, sig_v, ns_v, dist_v,
               idx_v2, sig_v2, ns_v2, dist_v2, c_v, sem, sem2):
    cid = lax.axis_index("c")
    sid = lax.axis_index("s")
    wid = sid * 2 + cid
    seg_elems = SEG * RPB

    def stage(p, idx_r, ns_r, dist_r, sig_r, sm):
        pltpu.sync_copy(idx_hbm.at[p], idx_r)
        pltpu.sync_copy(ns_hbm.at[p], ns_r)
        pltpu.sync_copy(dist_hbm.at[p], dist_r)
        mv = ns_r[pl.ds(0, 16)]
        for g in range(1, NGRP):
            mv = jnp.maximum(mv, ns_r[pl.ds(g * 16, 16)])
        m = mv[0]
        for l in range(1, 16):
            m = jnp.maximum(m, mv[l])
        nseg = (m + (SEG - 1)) >> 3

        def fire(b, _):
            pltpu.make_async_copy(
                table_ref.at[idx_r.at[pl.ds(b * seg_elems, seg_elems)]],
                sig_r.at[pl.ds(b * seg_elems, seg_elems)],
                sm,
            ).start()
            return 0

        lax.fori_loop(0, nseg, fire, 0)
        return m, nseg

    def finish(p, m, nseg, idx_r, ns_r, dist_r, sig_r, sm):
        def drain(b, _):
            pltpu.make_async_copy(
                table_ref.at[idx_r.at[pl.ds(b * seg_elems, seg_elems)]],
                sig_r.at[pl.ds(b * seg_elems, seg_elems)],
                sm,
            ).wait()
            return 0

        lax.fori_loop(0, nseg, drain, 0)

        nsg = [ns_r[pl.ds(g * 16, 16)] for g in range(NGRP)]
        dsg = [dist_r[pl.ds(g * 16, 16)] for g in range(NGRP)]
        ones = jnp.ones((16,), jnp.float32)

        def body(jj, carry):
            newc = []
            for g in range(NGRP):
                P, C = carry[2 * g], carry[2 * g + 1]
                sig = sig_r[pl.ds(jj * RPB + g * 16, 16)]
                valid = nsg[g] > jj
                s = jnp.maximum(sig, 0.0)
                e = jnp.exp(s * dsg[g])
                a = 1.0 - e
                om = jnp.where(valid, 1.0 - a, 1.0)
                P = P * om
                w = jnp.where(valid, a * P, 0.0)
                C = C * (1.0 + w)
                newc.append(P)
                newc.append(C)
            return tuple(newc)

        carry = lax.fori_loop(0, m, body, tuple([ones] * (2 * NGRP)))
        for g in range(NGRP):
            c_v[pl.ds(g * 16, 16)] = carry[2 * g + 1]
        pltpu.sync_copy(c_v, c_hbm.at[p])

    p1 = wid * 2
    p2 = wid * 2 + 1
    m1, n1 = stage(p1, idx_v, ns_v, dist_v, sig_v, sem)
    m2, n2 = stage(p2, idx_v2, ns_v2, dist_v2, sig_v2, sem2)
    finish(p1, m1, n1, idx_v, ns_v, dist_v, sig_v, sem)
    finish(p2, m2, n2, idx_v2, ns_v2, dist_v2, sig_v2, sem2)


@jax.jit
def kernel(w_sigma, w_rgb, rays):
    del w_rgb
    rays_t = rays.T

    idx, ns3, dist3 = pl.pallas_call(
        _tc_prepass,
        grid=(NBLK,),
        in_specs=[pl.BlockSpec((6, RPB), lambda p: (0, p))],
        out_specs=[
            pl.BlockSpec((1, CAP_S, RPB), lambda p: (p, 0, 0)),
            pl.BlockSpec((1, 1, RPB), lambda p: (p, 0, 0)),
            pl.BlockSpec((1, 1, RPB), lambda p: (p, 0, 0)),
        ],
        out_shape=[
            jax.ShapeDtypeStruct((NBLK, CAP_S, RPB), jnp.int32),
            jax.ShapeDtypeStruct((NBLK, 1, RPB), jnp.int32),
            jax.ShapeDtypeStruct((NBLK, 1, RPB), jnp.float32),
        ],
    )(rays_t)

    table = w_sigma.reshape(GRID * GRID * GRID)
    idx = idx.reshape(NBLK, CAP_S * RPB)
    ns2 = ns3.reshape(NBLK, RPB)
    dist2 = dist3.reshape(NBLK, RPB)

    c2 = pl.kernel(
        _sc_render,
        out_type=jax.ShapeDtypeStruct((NBLK, RPB), jnp.float32),
        mesh=plsc.VectorSubcoreMesh(core_axis_name="c", subcore_axis_name="s"),
        scratch_types=[
            pltpu.VMEM((CAP_S * RPB,), jnp.int32),
            pltpu.VMEM((CAP_S * RPB,), jnp.float32),
            pltpu.VMEM((RPB,), jnp.int32),
            pltpu.VMEM((RPB,), jnp.float32),
            pltpu.VMEM((CAP_S * RPB,), jnp.int32),
            pltpu.VMEM((CAP_S * RPB,), jnp.float32),
            pltpu.VMEM((RPB,), jnp.int32),
            pltpu.VMEM((RPB,), jnp.float32),
            pltpu.VMEM((RPB,), jnp.float32),
            pltpu.SemaphoreType.DMA,
            pltpu.SemaphoreType.DMA,
        ],
    )(table, idx, ns2, dist2)

    c = c2.reshape(N_RAYS)
    return jnp.stack([c, c, c, 1.0 - c], axis=1)


# speedup vs baseline: 285.7982x; 1.0003x over previous
import jax
import jax.numpy as jnp
from jax import lax
from jax.experimental import pallas as pl
from jax.experimental.pallas import tpu as pltpu
from jax.experimental.pallas import tpu_sc as plsc

GRID = 128
MAX_S = 256
N_RAYS = 8192
RPB = 128
NBLK = N_RAYS // RPB
SEG = 8
NGRP = RPB // 16
CAP_S = 168


def _tc_prepass(rays_t_ref, idx_ref, ns_ref, dist_ref):
    ox = rays_t_ref[0:1, :]
    oy = rays_t_ref[1:2, :]
    oz = rays_t_ref[2:3, :]
    dxr = rays_t_ref[3:4, :]
    dyr = rays_t_ref[4:5, :]
    dzr = rays_t_ref[5:6, :]
    nrm = jnp.sqrt(dxr * dxr + dyr * dyr + dzr * dzr)
    dx = dxr / nrm
    dy = dyr / nrm
    dz = dzr / nrm

    big = jnp.float32(1e30)

    def slab(o, d):
        zero = d == 0.0
        safe = jnp.where(zero, 1.0, d)
        i1 = jnp.where(zero, -big, (-1.5 - o) / safe)
        i2 = jnp.where(zero, big, (1.5 - o) / safe)
        nn = jnp.minimum(i1, i2)
        ff = jnp.maximum(i1, i2)
        okax = jnp.logical_or(~zero, (o >= -1.5) & (o <= 1.5))
        return nn, ff, okax

    nnx, ffx, okx = slab(ox, dx)
    nny, ffy, oky = slab(oy, dy)
    nnz, ffz, okz = slab(oz, dz)
    near = jnp.maximum(jnp.maximum(nnx, nny), nnz)
    far = jnp.minimum(jnp.minimum(ffx, ffy), ffz)
    ok = okx & oky & okz
    isect = (near <= far) & ok
    span = far - near
    ns = jnp.where(isect,
                   jnp.minimum(span * 32.0, 256.0).astype(jnp.int32),
                   jnp.int32(0))
    ns_f = jnp.maximum(ns, 1).astype(jnp.float32)
    dist = span / ns_f

    j = lax.broadcasted_iota(jnp.int32, (CAP_S, RPB), 0).astype(jnp.float32)
    t = near + span * (j + 0.5) / ns_f

    def axis_idx(o, d):
        pos = (o + d * t) / 1.5 * 0.5 + 0.5
        return jnp.clip(jnp.floor(pos * GRID).astype(jnp.int32), 0, GRID - 1)

    ix = axis_idx(ox, dx)
    iy = axis_idx(oy, dy)
    iz = axis_idx(oz, dz)
    idx = (ix * GRID + iy) * GRID + iz
    idx_ref[...] = idx.reshape(1, CAP_S, RPB)
    ns_ref[...] = ns.reshape(1, 1, RPB)
    dist_ref[...] = dist.reshape(1, 1, RPB)


def _sc_render(table_ref, idx_hbm, ns_hbm, dist_hbm, c_hbm,
               idx_v, sig_v, ns_v, dist_v,
               idx_v2, sig_v2, ns_v2, dist_v2, c_v, sem, sem2):
    cid = lax.axis_index("c")
    sid = lax.axis_index("s")
    wid = sid * 2 + cid
    seg_elems = SEG * RPB

    def stage(p, idx_r, ns_r, dist_r, sig_r, sm):
        pltpu.sync_copy(idx_hbm.at[p], idx_r)
        pltpu.sync_copy(ns_hbm.at[p], ns_r)
        pltpu.sync_copy(dist_hbm.at[p], dist_r)
        mv = ns_r[pl.ds(0, 16)]
        for g in range(1, NGRP):
            mv = jnp.maximum(mv, ns_r[pl.ds(g * 16, 16)])
        m = mv[0]
        for l in range(1, 16):
            m = jnp.maximum(m, mv[l])
        nseg = (m + (SEG - 1)) >> 3

        def fire(b, _):
            pltpu.make_async_copy(
                table_ref.at[idx_r.at[pl.ds(b * seg_elems, seg_elems)]],
                sig_r.at[pl.ds(b * seg_elems, seg_elems)],
                sm,
            ).start()
            return 0

        lax.fori_loop(0, nseg, fire, 0)
        return m, nseg

    def finish(p, m, nseg, idx_r, ns_r, dist_r, sig_r, sm):
        def drain(b, _):
            pltpu.make_async_copy(
                table_ref.at[idx_r.at[pl.ds(b * seg_elems, seg_elems)]],
                sig_r.at[pl.ds(b * seg_elems, seg_elems)],
                sm,
            ).wait()
            return 0

        lax.fori_loop(0, nseg, drain, 0)

        nsg = [ns_r[pl.ds(g * 16, 16)] for g in range(NGRP)]
        dsg = [dist_r[pl.ds(g * 16, 16)] for g in range(NGRP)]
        ones = jnp.ones((16,), jnp.float32)

        def body(jj, carry):
            newc = []
            for g in range(NGRP):
                P, C = carry[2 * g], carry[2 * g + 1]
                sig = sig_r[pl.ds(jj * RPB + g * 16, 16)]
                valid = nsg[g] > jj
                s = jnp.maximum(sig, 0.0)
                e = jnp.exp(s * dsg[g])
                a = 1.0 - e
                om = jnp.where(valid, 1.0 - a, 1.0)
                P = P * om
                w = jnp.where(valid, a * P, 0.0)
                C = C * (1.0 + w)
                newc.append(P)
                newc.append(C)
            return tuple(newc)

        carry = lax.fori_loop(0, m, body, tuple([ones] * (2 * NGRP)))
        for g in range(NGRP):
            c_v[pl.ds(g * 16, 16)] = carry[2 * g + 1]
        pltpu.sync_copy(c_v, c_hbm.at[p])

    p1 = wid * 2
    p2 = wid * 2 + 1
    m1, n1 = stage(p1, idx_v, ns_v, dist_v, sig_v, sem)
    m2, n2 = stage(p2, idx_v2, ns_v2, dist_v2, sig_v2, sem2)
    finish(p1, m1, n1, idx_v, ns_v, dist_v, sig_v, sem)
    finish(p2, m2, n2, idx_v2, ns_v2, dist_v2, sig_v2, sem2)


@jax.jit
def kernel(w_sigma, w_rgb, rays):
    del w_rgb
    rays_t = rays.T

    idx, ns3, dist3 = pl.pallas_call(
        _tc_prepass,
        grid=(NBLK,),
        in_specs=[pl.BlockSpec((6, RPB), lambda p: (0, p))],
        out_specs=[
            pl.BlockSpec((1, CAP_S, RPB), lambda p: (p, 0, 0)),
            pl.BlockSpec((1, 1, RPB), lambda p: (p, 0, 0)),
            pl.BlockSpec((1, 1, RPB), lambda p: (p, 0, 0)),
        ],
        out_shape=[
            jax.ShapeDtypeStruct((NBLK, CAP_S, RPB), jnp.int32),
            jax.ShapeDtypeStruct((NBLK, 1, RPB), jnp.int32),
            jax.ShapeDtypeStruct((NBLK, 1, RPB), jnp.float32),
        ],
    )(rays_t)

    table = w_sigma.reshape(GRID * GRID * GRID)
    idx = idx.reshape(NBLK, CAP_S * RPB)
    ns2 = ns3.reshape(NBLK, RPB)
    dist2 = dist3.reshape(NBLK, RPB)

    c2 = pl.kernel(
        _sc_render,
        out_type=jax.ShapeDtypeStruct((NBLK, RPB), jnp.float32),
        mesh=plsc.VectorSubcoreMesh(core_axis_name="c", subcore_axis_name="s"),
        scratch_types=[
            pltpu.VMEM((CAP_S * RPB,), jnp.int32),
            pltpu.VMEM((CAP_S * RPB,), jnp.float32),
            pltpu.VMEM((RPB,), jnp.int32),
            pltpu.VMEM((RPB,), jnp.float32),
            pltpu.VMEM((CAP_S * RPB,), jnp.int32),
            pltpu.VMEM((CAP_S * RPB,), jnp.float32),
            pltpu.VMEM((RPB,), jnp.int32),
            pltpu.VMEM((RPB,), jnp.float32),
            pltpu.VMEM((RPB,), jnp.float32),
            pltpu.SemaphoreType.DMA,
            pltpu.SemaphoreType.DMA,
        ],
    )(table, idx, ns2, dist2)

    c = c2.reshape(N_RAYS)
    return jnp.stack([c, c, c, 1.0 - c], axis=1)
